# Initial kernel scaffold; baseline (speedup 1.0000x reference)
import jax
import jax.numpy as jnp
from jax.experimental import pallas as pl


def kernel(theta):
    raise NotImplementedError("write your pallas kernel here")


# speedup vs baseline: 185.8467x; 185.8467x over previous
import jax
import jax.numpy as jnp
from jax.experimental import pallas as pl

_D = 256
_TMP = 1e-05
_MAX_ITER = 100


def _fw_body(theta_c_ref, theta_r_ref, out_ref):
    t_c = theta_c_ref[:] / jnp.float32(_TMP)
    t_r = theta_r_ref[:] / jnp.float32(_TMP)

    row_i = jax.lax.broadcasted_iota(jnp.int32, (_D, _D), 0)
    col_i = jax.lax.broadcasted_iota(jnp.int32, (_D, _D), 1)
    tie = col_i < row_i

    def lmo(g_c, g_r):
        lt = g_r < g_c
        a = jnp.where(lt | ((g_r == g_c) & tie), jnp.float32(1.0), jnp.float32(0.0))
        k_c = jnp.sum(a, axis=1, keepdims=True)
        colsum = jnp.sum(a, axis=0, keepdims=True)
        return jnp.float32(_D) - k_c, colsum + jnp.float32(1.0)

    mu_c0, mu_r0 = lmo(-t_c, -t_r)

    def cond(carry):
        it, done, _, _ = carry
        return (it < _MAX_ITER) & jnp.logical_not(done)

    def body(carry):
        it, _, mu_c, mu_r = carry
        g_c = mu_c - t_c
        g_r = mu_r - t_r
        s_c, s_r = lmo(g_c, g_r)
        d_c = s_c - mu_c
        d_r = s_r - mu_r
        denom = jnp.sum(d_c * d_c)
        num = -jnp.sum(g_c * d_c)
        gamma = jnp.where(denom > 0, jnp.clip(num / denom, 0.0, 1.0), jnp.float32(0.0))
        mu_c = mu_c + gamma * d_c
        mu_r = mu_r + gamma * d_r
        return it + 1, gamma <= 0.0, mu_c, mu_r

    _, _, _, mu_r = jax.lax.while_loop(
        cond, body, (jnp.int32(0), jnp.bool_(False), mu_c0, mu_r0)
    )
    out_ref[:] = mu_r


def kernel(theta):
    theta_c = theta.astype(jnp.float32).reshape(_D, 1)
    theta_r = theta_c.reshape(1, _D)
    out = pl.pallas_call(
        _fw_body,
        out_shape=jax.ShapeDtypeStruct((1, _D), jnp.float32),
    )(theta_c, theta_r)
    return out.reshape(_D)
